# baseline (speedup 1.0000x reference)
import jax
import jax.numpy as jnp
from jax import lax
from jax.experimental import pallas as pl
from jax.experimental.pallas import tpu as pltpu
from jax.experimental.pallas import tpu_sc as plsc

_VOCAB = 100000
_M = 10
_K = 256
_SUB = 30
_NTOK = 4096 * 50
_NW = 32
_TPW = _NTOK // _NW
_CH = 64
_NCH = _TPW // _CH
_SPB = 4
_BLK = _SPB * 50
_NBLK = _TPW // _BLK
_CBW = _M * _K * _SUB
_D = _M * _SUB


def _gather_body(ids_hbm, vec_hbm, codes_hbm, ids_v, c0, c1, c2, c3,
                 sg0, sg1, sg2, sg3, so0, so1, so2, so3):
    cid = lax.axis_index("c")
    sid = lax.axis_index("s")
    wid = sid * 2 + cid
    tok0 = wid * _TPW
    cb = (c0, c1, c2, c3)
    sg = (sg0, sg1, sg2, sg3)
    so = (so0, so1, so2, so3)

    pltpu.sync_copy(ids_hbm.at[pl.ds(tok0, _TPW)], ids_v)

    def issue(c, p):
        pltpu.async_copy(
            vec_hbm.at[ids_v.at[pl.ds(c * _CH, _CH)]], cb[p], sg[p])

    issue(0, 0)
    issue(1, 1)

    def step(c, p):
        q = (p + 2) % 4
        pltpu.make_async_copy(
            vec_hbm.at[ids_v.at[pl.ds(0, _CH)]], cb[p], sg[p]).wait()

        @pl.when((c >= 2) & (c + 2 < _NCH))
        def _():
            pltpu.make_async_copy(
                cb[q], codes_hbm.at[pl.ds(0, _CH)], so[q]).wait()

        @pl.when(c + 2 < _NCH)
        def _():
            issue(c + 2, q)

        pltpu.async_copy(
            cb[p], codes_hbm.at[pl.ds(tok0 + c * _CH, _CH)], so[p])

    def quad(cq, carry):
        for p in range(4):
            step(cq * 4 + p, p)
        return carry

    lax.fori_loop(0, _NCH // 4, quad, 0)
    for p in range(4):
        pltpu.make_async_copy(
            cb[p], codes_hbm.at[pl.ds(0, _CH)], so[p]).wait()


def _assemble_body(codes_hbm, cw_hbm, out_hbm,
                   cb_v, cod0, cod1, out_v, sc0, sc1, so0, so1):
    cid = lax.axis_index("c")
    sid = lax.axis_index("s")
    wid = sid * 2 + cid
    tok0 = wid * _TPW
    sent0 = wid * (_TPW // 50)
    lane = lax.iota(jnp.int32, 16)
    codb, scb = (cod0, cod1), (sc0, sc1)

    pltpu.sync_copy(cw_hbm, cb_v)
    pltpu.async_copy(codes_hbm.at[pl.ds(tok0 * 16, _BLK * 16)], cod0, sc0)

    def group(blk, pb, kp, j, cod_ref):
        k = kp * 2 + j
        act = jnp.where((k & 3) == 3, 2, 16)
        m = lane < act
        base = pb * 100 + (k >> 2) * 50 + (k & 3) * 16
        slot = kp >> 1
        s_loc = blk * _SPB + pb * 2 + (k >> 2)

        if j == 0:
            @pl.when((kp == 0) & (s_loc >= 2))
            def _():
                pltpu.make_async_copy(
                    out_v.at[pl.ds(0, 1)],
                    out_hbm.at[pl.ds(sent0, 1)], so0).wait()

            @pl.when((kp == 2) & (s_loc >= 2))
            def _():
                pltpu.make_async_copy(
                    out_v.at[pl.ds(1, 1)],
                    out_hbm.at[pl.ds(sent0, 1)], so1).wait()

        crow = (base + lane) * 16
        lvec = (k & 3) * 16 + lane
        slotv = jnp.full((16,), slot, jnp.int32)
        for i in range(_M):
            c16 = plsc.load_gather(cod_ref, [crow + i], mask=m) & 255
            bvec = c16 * _SUB + i * (_K * _SUB)
            col0 = jnp.full((16,), i * _SUB, jnp.int32)
            for d0 in range(0, _SUB, 6):
                vs = [plsc.load_gather(cb_v, [bvec + (d0 + dd)])
                      for dd in range(6)]
                for dd in range(6):
                    plsc.store_scatter(
                        out_v, [slotv, lvec, col0 + (d0 + dd)], vs[dd],
                        mask=m)

        if j == 1:
            @pl.when(kp == 1)
            def _():
                pltpu.async_copy(out_v.at[pl.ds(0, 1)],
                                 out_hbm.at[pl.ds(sent0 + s_loc, 1)], so0)

            @pl.when(kp == 3)
            def _():
                pltpu.async_copy(out_v.at[pl.ds(1, 1)],
                                 out_hbm.at[pl.ds(sent0 + s_loc, 1)], so1)

    def block(blk, bi):
        pltpu.make_async_copy(
            codes_hbm.at[pl.ds(tok0 * 16, _BLK * 16)], codb[bi],
            scb[bi]).wait()

        @pl.when(blk + 1 < _NBLK)
        def _():
            pltpu.async_copy(
                codes_hbm.at[pl.ds((tok0 + (blk + 1) * _BLK) * 16, _BLK * 16)],
                codb[1 - bi], scb[1 - bi])

        def q_body(q, carry):
            pb = q >> 2
            kp = q & 3
            group(blk, pb, kp, 0, codb[bi])
            group(blk, pb, kp, 1, codb[bi])
            return carry

        lax.fori_loop(0, _SPB * 2, q_body, 0)

    def bp_body(bp, carry):
        block(bp * 2, 0)
        block(bp * 2 + 1, 1)
        return carry

    lax.fori_loop(0, _NBLK // 2, bp_body, 0)

    pltpu.make_async_copy(out_v.at[pl.ds(0, 1)],
                          out_hbm.at[pl.ds(sent0, 1)], so0).wait()
    pltpu.make_async_copy(out_v.at[pl.ds(1, 1)],
                          out_hbm.at[pl.ds(sent0, 1)], so1).wait()


def kernel(input_ids, codewords, vectors):
    ids = input_ids.reshape(_NTOK)
    cw = codewords.reshape(_CBW)
    vec16 = jnp.pad(vectors, ((0, 0), (0, 16 - _M)))
    mesh = plsc.VectorSubcoreMesh(core_axis_name="c", subcore_axis_name="s")
    codes = pl.kernel(
        _gather_body,
        out_type=jax.ShapeDtypeStruct((_NTOK, 16), jnp.int32),
        mesh=mesh,
        compiler_params=pltpu.CompilerParams(
            use_tc_tiling_on_sc=False, needs_layout_passes=False),
        scratch_types=[
            pltpu.VMEM((_TPW,), jnp.int32),
            pltpu.VMEM((_CH, 16), jnp.int32),
            pltpu.VMEM((_CH, 16), jnp.int32),
            pltpu.VMEM((_CH, 16), jnp.int32),
            pltpu.VMEM((_CH, 16), jnp.int32),
        ] + [pltpu.SemaphoreType.DMA] * 8,
    )(ids, vec16)
    out = pl.kernel(
        _assemble_body,
        out_type=jax.ShapeDtypeStruct((4096, 50, _D), jnp.float32),
        mesh=mesh,
        compiler_params=pltpu.CompilerParams(
            use_tc_tiling_on_sc=True, needs_layout_passes=False),
        scratch_types=[
            pltpu.VMEM((_CBW,), jnp.float32),
            pltpu.VMEM((_BLK * 16,), jnp.int32),
            pltpu.VMEM((_BLK * 16,), jnp.int32),
            pltpu.VMEM((2, 50, _D), jnp.float32),
        ] + [pltpu.SemaphoreType.DMA] * 4,
    )(codes.reshape(_NTOK * 16), cw)
    return out


# speedup vs baseline: 1.6265x; 1.6265x over previous
import jax
import jax.numpy as jnp
from jax import lax
from jax.experimental import pallas as pl
from jax.experimental.pallas import tpu as pltpu
from jax.experimental.pallas import tpu_sc as plsc

_VOCAB = 100000
_M = 10
_K = 256
_SUB = 30
_NTOK = 4096 * 50
_NW = 32
_TPW = _NTOK // _NW
_CH = 64
_NCH = _TPW // _CH
_SPB = 4
_BLK = _SPB * 50
_NBLK = _TPW // _BLK
_CBW = _M * _K * _SUB
_D = _M * _SUB


def _gather_body(ids_hbm, vec_hbm, codes_hbm, ids_v, c0, c1, c2, c3,
                 sg0, sg1, sg2, sg3, so0, so1, so2, so3):
    cid = lax.axis_index("c")
    sid = lax.axis_index("s")
    wid = sid * 2 + cid
    tok0 = wid * _TPW
    cb = (c0, c1, c2, c3)
    sg = (sg0, sg1, sg2, sg3)
    so = (so0, so1, so2, so3)

    pltpu.sync_copy(ids_hbm.at[pl.ds(tok0, _TPW)], ids_v)

    def issue(c, p):
        pltpu.async_copy(
            vec_hbm.at[ids_v.at[pl.ds(c * _CH, _CH)]], cb[p], sg[p])

    issue(0, 0)
    issue(1, 1)

    def step(c, p):
        q = (p + 2) % 4
        pltpu.make_async_copy(
            vec_hbm.at[ids_v.at[pl.ds(0, _CH)]], cb[p], sg[p]).wait()

        @pl.when((c >= 2) & (c + 2 < _NCH))
        def _():
            pltpu.make_async_copy(
                cb[q], codes_hbm.at[pl.ds(0, _CH)], so[q]).wait()

        @pl.when(c + 2 < _NCH)
        def _():
            issue(c + 2, q)

        pltpu.async_copy(
            cb[p], codes_hbm.at[pl.ds(tok0 + c * _CH, _CH)], so[p])

    def quad(cq, carry):
        for p in range(4):
            step(cq * 4 + p, p)
        return carry

    lax.fori_loop(0, _NCH // 4, quad, 0)
    for p in range(4):
        pltpu.make_async_copy(
            cb[p], codes_hbm.at[pl.ds(0, _CH)], so[p]).wait()


def _assemble_body(codes_hbm, cw_hbm, out_hbm,
                   cb_v, cod0, cod1, out_v, sc0, sc1, so0, so1):
    cid = lax.axis_index("c")
    sid = lax.axis_index("s")
    wid = sid * 2 + cid
    tok0 = wid * _TPW
    sent0 = wid * (_TPW // 50)
    lane = lax.iota(jnp.int32, 16)
    codb, scb = (cod0, cod1), (sc0, sc1)

    pltpu.sync_copy(cw_hbm, cb_v)
    pltpu.async_copy(codes_hbm.at[pl.ds(tok0 * 16, _BLK * 16)], cod0, sc0)

    def group(blk, pb, kp, j, cod_ref):
        k = kp * 2 + j
        act = jnp.where((k & 3) == 3, 2, 16)
        m = lane < act
        base = pb * 100 + (k >> 2) * 50 + (k & 3) * 16
        slot = kp >> 1
        s_loc = blk * _SPB + pb * 2 + (k >> 2)

        if j == 0:
            @pl.when((kp == 0) & (s_loc >= 2))
            def _():
                pltpu.make_async_copy(
                    out_v.at[pl.ds(0, 1)],
                    out_hbm.at[pl.ds(sent0, 1)], so0).wait()

            @pl.when((kp == 2) & (s_loc >= 2))
            def _():
                pltpu.make_async_copy(
                    out_v.at[pl.ds(1, 1)],
                    out_hbm.at[pl.ds(sent0, 1)], so1).wait()

        crow = (base + lane) * 16
        lvec = (k & 3) * 16 + lane
        slotv = jnp.full((16,), slot, jnp.int32)
        for i in range(_M):
            c16 = plsc.load_gather(cod_ref, [crow + i], mask=m) & 255
            bvec = c16 * _SUB + i * (_K * _SUB)
            col0 = jnp.full((16,), i * _SUB, jnp.int32)

            @plsc.parallel_loop(0, _SUB, unroll=6)
            def _dl(d):
                vals = plsc.load_gather(cb_v, [bvec + d])
                plsc.store_scatter(out_v, [slotv, lvec, col0 + d], vals,
                                   mask=m)

        if j == 1:
            @pl.when(kp == 1)
            def _():
                pltpu.async_copy(out_v.at[pl.ds(0, 1)],
                                 out_hbm.at[pl.ds(sent0 + s_loc, 1)], so0)

            @pl.when(kp == 3)
            def _():
                pltpu.async_copy(out_v.at[pl.ds(1, 1)],
                                 out_hbm.at[pl.ds(sent0 + s_loc, 1)], so1)

    def block(blk, bi):
        pltpu.make_async_copy(
            codes_hbm.at[pl.ds(tok0 * 16, _BLK * 16)], codb[bi],
            scb[bi]).wait()

        @pl.when(blk + 1 < _NBLK)
        def _():
            pltpu.async_copy(
                codes_hbm.at[pl.ds((tok0 + (blk + 1) * _BLK) * 16, _BLK * 16)],
                codb[1 - bi], scb[1 - bi])

        def q_body(q, carry):
            pb = q >> 2
            kp = q & 3
            group(blk, pb, kp, 0, codb[bi])
            group(blk, pb, kp, 1, codb[bi])
            return carry

        lax.fori_loop(0, _SPB * 2, q_body, 0)

    def bp_body(bp, carry):
        block(bp * 2, 0)
        block(bp * 2 + 1, 1)
        return carry

    lax.fori_loop(0, _NBLK // 2, bp_body, 0)

    pltpu.make_async_copy(out_v.at[pl.ds(0, 1)],
                          out_hbm.at[pl.ds(sent0, 1)], so0).wait()
    pltpu.make_async_copy(out_v.at[pl.ds(1, 1)],
                          out_hbm.at[pl.ds(sent0, 1)], so1).wait()


def kernel(input_ids, codewords, vectors):
    ids = input_ids.reshape(_NTOK)
    cw = codewords.reshape(_CBW)
    vec16 = jnp.pad(vectors, ((0, 0), (0, 16 - _M)))
    mesh = plsc.VectorSubcoreMesh(core_axis_name="c", subcore_axis_name="s")
    codes = pl.kernel(
        _gather_body,
        out_type=jax.ShapeDtypeStruct((_NTOK, 16), jnp.int32),
        mesh=mesh,
        compiler_params=pltpu.CompilerParams(
            use_tc_tiling_on_sc=False, needs_layout_passes=False),
        scratch_types=[
            pltpu.VMEM((_TPW,), jnp.int32),
            pltpu.VMEM((_CH, 16), jnp.int32),
            pltpu.VMEM((_CH, 16), jnp.int32),
            pltpu.VMEM((_CH, 16), jnp.int32),
            pltpu.VMEM((_CH, 16), jnp.int32),
        ] + [pltpu.SemaphoreType.DMA] * 8,
    )(ids, vec16)
    out = pl.kernel(
        _assemble_body,
        out_type=jax.ShapeDtypeStruct((4096, 50, _D), jnp.float32),
        mesh=mesh,
        compiler_params=pltpu.CompilerParams(
            use_tc_tiling_on_sc=False, needs_layout_passes=False),
        scratch_types=[
            pltpu.VMEM((_CBW,), jnp.float32),
            pltpu.VMEM((_BLK * 16,), jnp.int32),
            pltpu.VMEM((_BLK * 16,), jnp.int32),
            pltpu.VMEM((2, 50, _D), jnp.float32),
        ] + [pltpu.SemaphoreType.DMA] * 4,
    )(codes.reshape(_NTOK * 16), cw)
    return out
